# baseline (speedup 1.0000x reference)
import jax
import jax.numpy as jnp
from jax import lax
from jax.experimental import pallas as pl
from jax.experimental.pallas import tpu as pltpu

_CONF_THRES = 0.25
_IOU_THRES = 0.45
_MAX_DET = 1000
_MAX_WH = 4096.0

_N = 20000
_NPAD = 20480
_ROWS = 160
_COLS = 128
_NCLS = 80
_NIMG = 4
_S = 512
_TRASH = _NCLS * _S

_NEG_INF = float("-inf")


def _shift_lanes(x, k):
    return jnp.concatenate(
        [jnp.zeros((x.shape[0], k), x.dtype), x[:, :-k]], axis=1)


def _shift_rows(x, k):
    return jnp.concatenate(
        [jnp.zeros((k, x.shape[1]), x.dtype), x[:-k, :]], axis=0)


def _score_body(p_ref, rows_ref, dst_ref):
    li = (lax.broadcasted_iota(jnp.int32, (_ROWS, _COLS), 0) * _COLS
          + lax.broadcasted_iota(jnp.int32, (_ROWS, _COLS), 1))
    for b in range(_NIMG):
        cx = p_ref[b, 0]
        cy = p_ref[b, 1]
        w = p_ref[b, 2]
        h = p_ref[b, 3]
        obj = p_ref[b, 4]
        x1 = cx - w / 2
        y1 = cy - h / 2
        x2 = cx + w / 2
        y2 = cy + h / 2
        best = p_ref[b, 5] * obj
        jbest = jnp.zeros((_ROWS, _COLS), jnp.int32)
        for c in range(1, _NCLS):
            v = p_ref[b, 5 + c] * obj
            take = v > best
            jbest = jnp.where(take, c, jbest)
            best = jnp.maximum(best, v)
        conf = best
        valid = (obj > _CONF_THRES) & (conf > _CONF_THRES)

        slot = jnp.zeros((_ROWS, _COLS), jnp.int32)
        for c in range(_NCLS):
            m = (valid & (jbest == c)).astype(jnp.int32)
            s = m
            for k in (1, 2, 4, 8, 16, 32, 64):
                s = s + _shift_lanes(s, k)
            excl_lane = s - m
            rowtot = s[:, _COLS - 1:_COLS]
            t = rowtot
            for k in (1, 2, 4, 8, 16, 32, 64, 128):
                t = t + _shift_rows(t, k)
            excl_row = t - rowtot
            slot = jnp.where(jbest == c, excl_row + excl_lane, slot)

        dst = jnp.where(valid & (slot < _S), jbest * _S + slot, _TRASH)
        rows_ref[b, 0] = conf
        rows_ref[b, 1] = x1
        rows_ref[b, 2] = y1
        rows_ref[b, 3] = x2
        rows_ref[b, 4] = y2
        rows_ref[b, 5] = li.astype(jnp.float32)
        dst_ref[b] = dst


def _merge_body(colsc_ref, colp_ref, out_ref, sg_ref):
    lane_s = lax.broadcasted_iota(jnp.int32, (1, _S), 1)
    lane_cb = lax.broadcasted_iota(jnp.int32, (_NIMG, _COLS), 1)
    lane_sb = lax.broadcasted_iota(jnp.int32, (_NIMG, _S), 1)
    lane6 = lax.broadcasted_iota(jnp.int32, (1, 6), 1)
    big = jnp.float32(3.0e38)
    bigi = jnp.int32(2 ** 30)

    for b in range(_NIMG):
        sg_ref[b] = colsc_ref[b]

    hs0 = jnp.full((_NIMG, _COLS), _NEG_INF, jnp.float32)
    hslot0 = jnp.zeros((_NIMG, _COLS), jnp.float32)
    hidx0 = jnp.zeros((_NIMG, _COLS), jnp.float32)
    for c in range(_NCLS):
        srows = colsc_ref[:, c, :]
        m = jnp.max(srows, axis=1, keepdims=True)
        sl = jnp.min(jnp.where(srows == m, lane_sb, bigi),
                     axis=1, keepdims=True)
        idxr = colp_ref[:, 8 * c + 4, :]
        ii = jnp.sum(jnp.where(lane_sb == sl, idxr, 0.0),
                     axis=1, keepdims=True)
        upd = lane_cb == c
        hs0 = jnp.where(upd, m, hs0)
        hslot0 = jnp.where(upd, sl.astype(jnp.float32), hslot0)
        hidx0 = jnp.where(upd, ii, hidx0)

    def step(t, carry):
        hs, hslot, hidx = carry
        m4 = jnp.max(hs, axis=1, keepdims=True)
        cand = hs == m4
        mi4 = jnp.min(jnp.where(cand, hidx, big), axis=1, keepdims=True)
        csel = cand & (hidx == mi4)
        cstar4 = jnp.min(jnp.where(csel, lane_cb, bigi),
                         axis=1, keepdims=True)
        slot4 = jnp.sum(jnp.where(csel, hslot, 0.0),
                        axis=1, keepdims=True)
        ok4 = m4 > 0.0

        hs_rows = []
        hsl_rows = []
        hix_rows = []
        for b in range(_NIMG):
            cstar_v = cstar4[b:b + 1]
            cstar_s = jnp.min(cstar_v)
            okb = ok4[b:b + 1]
            mb = m4[b:b + 1]
            cstar_f = cstar_v.astype(jnp.float32)
            slot_i = slot4[b:b + 1].astype(jnp.int32)

            rows5 = colp_ref[b, pl.ds(cstar_s * 8, 8), :]
            selv = lane_s == slot_i
            w5 = jnp.sum(jnp.where(selv, rows5, 0.0),
                         axis=1, keepdims=True)
            wx1 = w5[0:1]
            wy1 = w5[1:2]
            wx2 = w5[2:3]
            wy2 = w5[3:4]

            row = jnp.where(lane6 == 0, wx1,
                  jnp.where(lane6 == 1, wy1,
                  jnp.where(lane6 == 2, wx2,
                  jnp.where(lane6 == 3, wy2,
                  jnp.where(lane6 == 4, mb, cstar_f)))))
            row = jnp.where(okb, row, 0.0)
            out_ref[b, pl.ds(t, 1), :] = row

            offs = cstar_f * _MAX_WH
            x1r = rows5[0:1]
            y1r = rows5[1:2]
            x2r = rows5[2:3]
            y2r = rows5[3:4]
            idxr = rows5[4:5]
            wbx1 = wx1 + offs
            wby1 = wy1 + offs
            wbx2 = wx2 + offs
            wby2 = wy2 + offs
            bx1 = x1r + offs
            by1 = y1r + offs
            bx2 = x2r + offs
            by2 = y2r + offs
            xx1 = jnp.maximum(wbx1, bx1)
            yy1 = jnp.maximum(wby1, by1)
            xx2 = jnp.minimum(wbx2, bx2)
            yy2 = jnp.minimum(wby2, by2)
            inter = (jnp.maximum(xx2 - xx1, 0.0)
                     * jnp.maximum(yy2 - yy1, 0.0))
            a1 = (wbx2 - wbx1) * (wby2 - wby1)
            a2 = (bx2 - bx1) * (by2 - by1)
            iou = inter / (a1 + a2 - inter + 1e-7)
            srow = sg_ref[b, pl.ds(cstar_s, 1), :]
            srow2 = jnp.where(iou > _IOU_THRES, _NEG_INF, srow)
            srow2 = jnp.where(selv, _NEG_INF, srow2)
            srow_new = jnp.where(okb, srow2, srow)
            sg_ref[b, pl.ds(cstar_s, 1), :] = srow_new

            m2 = jnp.max(srow_new, axis=1, keepdims=True)
            sl2 = jnp.min(jnp.where(srow_new == m2, lane_s, bigi),
                          axis=1, keepdims=True)
            ii2 = jnp.sum(jnp.where(lane_s == sl2, idxr, 0.0),
                          axis=1, keepdims=True)
            upd = (lane_cb[b:b + 1] == cstar_v) & okb
            hs_rows.append(jnp.where(upd, m2, hs[b:b + 1]))
            hsl_rows.append(
                jnp.where(upd, sl2.astype(jnp.float32), hslot[b:b + 1]))
            hix_rows.append(jnp.where(upd, ii2, hidx[b:b + 1]))
        return (jnp.concatenate(hs_rows, axis=0),
                jnp.concatenate(hsl_rows, axis=0),
                jnp.concatenate(hix_rows, axis=0))

    lax.fori_loop(0, _MAX_DET, step, (hs0, hslot0, hidx0))


def kernel(x):
    pred = x[0]
    pad = jnp.zeros((_NIMG, _NPAD - _N, pred.shape[-1]), pred.dtype)
    p = jnp.concatenate([pred, pad], axis=1)
    pt = p.reshape(_NIMG, _ROWS, _COLS, pred.shape[-1]).transpose(0, 3, 1, 2)

    rows, dst = pl.pallas_call(
        _score_body,
        out_shape=(
            jax.ShapeDtypeStruct((_NIMG, 6, _ROWS, _COLS), jnp.float32),
            jax.ShapeDtypeStruct((_NIMG, _ROWS, _COLS), jnp.int32),
        ),
    )(pt)

    flat_rows = rows.transpose(0, 2, 3, 1).reshape(_NIMG, _NPAD, 6)
    dstf = dst.reshape(_NIMG, _NPAD)
    col = jnp.zeros((_NIMG, _TRASH + 1, 6), jnp.float32)
    col = col.at[jnp.arange(_NIMG)[:, None], dstf].set(flat_rows)
    colsc = col[:, :_TRASH, 0].reshape(_NIMG, _NCLS, _S)
    colp5 = (col[:, :_TRASH, 1:6]
             .reshape(_NIMG, _NCLS, _S, 5)
             .transpose(0, 1, 3, 2))
    colp = jnp.concatenate(
        [colp5, jnp.zeros((_NIMG, _NCLS, 3, _S), jnp.float32)],
        axis=2).reshape(_NIMG, _NCLS * 8, _S)

    out = pl.pallas_call(
        _merge_body,
        out_shape=jax.ShapeDtypeStruct((_NIMG, _MAX_DET, 6), jnp.float32),
        scratch_shapes=[pltpu.VMEM((_NIMG, _NCLS, _S), jnp.float32)],
    )(colsc, colp)
    return out


# speedup vs baseline: 1.2199x; 1.2199x over previous
import jax
import jax.numpy as jnp
from jax import lax
from jax.experimental import pallas as pl
from jax.experimental.pallas import tpu as pltpu

_CONF_THRES = 0.25
_IOU_THRES = 0.45
_MAX_DET = 1000
_MAX_WH = 4096.0

_N = 20000
_NPAD = 20480
_ROWS = 160
_COLS = 128
_NCLS = 80
_NIMG = 4
_S = 512
_TRASH = _NCLS * _S

_NEG_INF = float("-inf")


def _shift_lanes(x, k):
    return jnp.concatenate(
        [jnp.zeros((x.shape[0], k), x.dtype), x[:, :-k]], axis=1)


def _shift_rows(x, k):
    return jnp.concatenate(
        [jnp.zeros((k, x.shape[1]), x.dtype), x[:-k, :]], axis=0)


def _score_body(p_ref, rows_ref, dst_ref):
    li = (lax.broadcasted_iota(jnp.int32, (_ROWS, _COLS), 0) * _COLS
          + lax.broadcasted_iota(jnp.int32, (_ROWS, _COLS), 1))
    for b in range(_NIMG):
        cx = p_ref[b, 0]
        cy = p_ref[b, 1]
        w = p_ref[b, 2]
        h = p_ref[b, 3]
        obj = p_ref[b, 4]
        x1 = cx - w / 2
        y1 = cy - h / 2
        x2 = cx + w / 2
        y2 = cy + h / 2
        best = p_ref[b, 5] * obj
        jbest = jnp.zeros((_ROWS, _COLS), jnp.int32)
        for c in range(1, _NCLS):
            v = p_ref[b, 5 + c] * obj
            take = v > best
            jbest = jnp.where(take, c, jbest)
            best = jnp.maximum(best, v)
        conf = best
        valid = (obj > _CONF_THRES) & (conf > _CONF_THRES)

        ustrict = (lax.broadcasted_iota(jnp.int32, (_COLS, _COLS), 0)
                   < lax.broadcasted_iota(jnp.int32, (_COLS, _COLS), 1)
                   ).astype(jnp.float32)
        tstrict = (lax.broadcasted_iota(jnp.int32, (_ROWS, _ROWS), 1)
                   < lax.broadcasted_iota(jnp.int32, (_ROWS, _ROWS), 0)
                   ).astype(jnp.float32)
        slot_lane = jnp.zeros((_ROWS, _COLS), jnp.float32)
        rowtots = []
        for c in range(_NCLS):
            m = jnp.where(valid & (jbest == c), 1.0, 0.0)
            excl_lane = jnp.dot(m, ustrict)
            slot_lane = slot_lane + jnp.where(jbest == c, excl_lane, 0.0)
            rowtots.append(jnp.sum(m, axis=1, keepdims=True))
        rmat = jnp.concatenate(rowtots, axis=1)
        excl_row_all = jnp.dot(tstrict, rmat)
        slot_row = jnp.zeros((_ROWS, _COLS), jnp.float32)
        for c in range(_NCLS):
            slot_row = slot_row + jnp.where(
                jbest == c, excl_row_all[:, c:c + 1], 0.0)
        slot = (slot_lane + slot_row).astype(jnp.int32)

        dst = jnp.where(valid & (slot < _S), jbest * _S + slot, _TRASH)
        rows_ref[b, 0] = conf
        rows_ref[b, 1] = x1
        rows_ref[b, 2] = y1
        rows_ref[b, 3] = x2
        rows_ref[b, 4] = y2
        rows_ref[b, 5] = li.astype(jnp.float32)
        dst_ref[b] = dst


def _merge_body(colsc_ref, colp_ref, out_ref, sg_ref):
    lane_s = lax.broadcasted_iota(jnp.int32, (1, _S), 1)
    lane_cb = lax.broadcasted_iota(jnp.int32, (_NIMG, _COLS), 1)
    lane_sb = lax.broadcasted_iota(jnp.int32, (_NIMG, _S), 1)
    lane6 = lax.broadcasted_iota(jnp.int32, (1, 6), 1)
    big = jnp.float32(3.0e38)
    bigi = jnp.int32(2 ** 30)

    for b in range(_NIMG):
        sg_ref[b] = colsc_ref[b]

    hs0 = jnp.full((_NIMG, _COLS), _NEG_INF, jnp.float32)
    hslot0 = jnp.zeros((_NIMG, _COLS), jnp.float32)
    hidx0 = jnp.zeros((_NIMG, _COLS), jnp.float32)
    for c in range(_NCLS):
        srows = colsc_ref[:, c, :]
        m = jnp.max(srows, axis=1, keepdims=True)
        sl = jnp.min(jnp.where(srows == m, lane_sb, bigi),
                     axis=1, keepdims=True)
        idxr = colp_ref[:, 8 * c + 4, :]
        ii = jnp.sum(jnp.where(lane_sb == sl, idxr, 0.0),
                     axis=1, keepdims=True)
        upd = lane_cb == c
        hs0 = jnp.where(upd, m, hs0)
        hslot0 = jnp.where(upd, sl.astype(jnp.float32), hslot0)
        hidx0 = jnp.where(upd, ii, hidx0)

    def step(t, carry):
        hs, hslot, hidx = carry
        m4 = jnp.max(hs, axis=1, keepdims=True)
        cand = hs == m4
        mi4 = jnp.min(jnp.where(cand, hidx, big), axis=1, keepdims=True)
        csel = cand & (hidx == mi4)
        cstar4 = jnp.min(jnp.where(csel, lane_cb, bigi),
                         axis=1, keepdims=True)
        slot4 = jnp.sum(jnp.where(csel, hslot, 0.0),
                        axis=1, keepdims=True)
        ok4 = m4 > 0.0

        hs_rows = []
        hsl_rows = []
        hix_rows = []
        for b in range(_NIMG):
            cstar_v = cstar4[b:b + 1]
            cstar_s = jnp.min(cstar_v)
            okb = ok4[b:b + 1]
            mb = m4[b:b + 1]
            cstar_f = cstar_v.astype(jnp.float32)
            slot_i = slot4[b:b + 1].astype(jnp.int32)

            rows5 = colp_ref[b, pl.ds(cstar_s * 8, 8), :]
            selv = lane_s == slot_i
            w5 = jnp.sum(jnp.where(selv, rows5, 0.0),
                         axis=1, keepdims=True)
            wx1 = w5[0:1]
            wy1 = w5[1:2]
            wx2 = w5[2:3]
            wy2 = w5[3:4]

            row = jnp.where(lane6 == 0, wx1,
                  jnp.where(lane6 == 1, wy1,
                  jnp.where(lane6 == 2, wx2,
                  jnp.where(lane6 == 3, wy2,
                  jnp.where(lane6 == 4, mb, cstar_f)))))
            row = jnp.where(okb, row, 0.0)
            out_ref[b, pl.ds(t, 1), :] = row

            offs = cstar_f * _MAX_WH
            x1r = rows5[0:1]
            y1r = rows5[1:2]
            x2r = rows5[2:3]
            y2r = rows5[3:4]
            idxr = rows5[4:5]
            wbx1 = wx1 + offs
            wby1 = wy1 + offs
            wbx2 = wx2 + offs
            wby2 = wy2 + offs
            bx1 = x1r + offs
            by1 = y1r + offs
            bx2 = x2r + offs
            by2 = y2r + offs
            xx1 = jnp.maximum(wbx1, bx1)
            yy1 = jnp.maximum(wby1, by1)
            xx2 = jnp.minimum(wbx2, bx2)
            yy2 = jnp.minimum(wby2, by2)
            inter = (jnp.maximum(xx2 - xx1, 0.0)
                     * jnp.maximum(yy2 - yy1, 0.0))
            a1 = (wbx2 - wbx1) * (wby2 - wby1)
            a2 = (bx2 - bx1) * (by2 - by1)
            iou = inter / (a1 + a2 - inter + 1e-7)
            srow = sg_ref[b, pl.ds(cstar_s, 1), :]
            srow2 = jnp.where(iou > _IOU_THRES, _NEG_INF, srow)
            srow2 = jnp.where(selv, _NEG_INF, srow2)
            srow_new = jnp.where(okb, srow2, srow)
            sg_ref[b, pl.ds(cstar_s, 1), :] = srow_new

            m2 = jnp.max(srow_new, axis=1, keepdims=True)
            sl2 = jnp.min(jnp.where(srow_new == m2, lane_s, bigi),
                          axis=1, keepdims=True)
            ii2 = jnp.sum(jnp.where(lane_s == sl2, idxr, 0.0),
                          axis=1, keepdims=True)
            upd = (lane_cb[b:b + 1] == cstar_v) & okb
            hs_rows.append(jnp.where(upd, m2, hs[b:b + 1]))
            hsl_rows.append(
                jnp.where(upd, sl2.astype(jnp.float32), hslot[b:b + 1]))
            hix_rows.append(jnp.where(upd, ii2, hidx[b:b + 1]))
        return (jnp.concatenate(hs_rows, axis=0),
                jnp.concatenate(hsl_rows, axis=0),
                jnp.concatenate(hix_rows, axis=0))

    lax.fori_loop(0, _MAX_DET, step, (hs0, hslot0, hidx0))


def kernel(x):
    pred = x[0]
    pad = jnp.zeros((_NIMG, _NPAD - _N, pred.shape[-1]), pred.dtype)
    p = jnp.concatenate([pred, pad], axis=1)
    pt = p.reshape(_NIMG, _ROWS, _COLS, pred.shape[-1]).transpose(0, 3, 1, 2)

    rows, dst = pl.pallas_call(
        _score_body,
        out_shape=(
            jax.ShapeDtypeStruct((_NIMG, 6, _ROWS, _COLS), jnp.float32),
            jax.ShapeDtypeStruct((_NIMG, _ROWS, _COLS), jnp.int32),
        ),
    )(pt)

    flat_rows = rows.transpose(0, 2, 3, 1).reshape(_NIMG, _NPAD, 6)
    dstf = dst.reshape(_NIMG, _NPAD)
    col = jnp.zeros((_NIMG, _TRASH + 1, 6), jnp.float32)
    col = col.at[jnp.arange(_NIMG)[:, None], dstf].set(flat_rows)
    colsc = col[:, :_TRASH, 0].reshape(_NIMG, _NCLS, _S)
    colp5 = (col[:, :_TRASH, 1:6]
             .reshape(_NIMG, _NCLS, _S, 5)
             .transpose(0, 1, 3, 2))
    colp = jnp.concatenate(
        [colp5, jnp.zeros((_NIMG, _NCLS, 3, _S), jnp.float32)],
        axis=2).reshape(_NIMG, _NCLS * 8, _S)

    out = pl.pallas_call(
        _merge_body,
        out_shape=jax.ShapeDtypeStruct((_NIMG, _MAX_DET, 6), jnp.float32),
        scratch_shapes=[pltpu.VMEM((_NIMG, _NCLS, _S), jnp.float32)],
    )(colsc, colp)
    return out


# speedup vs baseline: 1.8107x; 1.4843x over previous
import jax
import jax.numpy as jnp
from jax import lax
from jax.experimental import pallas as pl
from jax.experimental.pallas import tpu as pltpu
from jax.experimental.pallas import tpu_sc as plsc
import functools

_CONF_THRES = 0.25
_IOU_THRES = 0.45
_MAX_DET = 1000
_MAX_WH = 4096.0

_N = 20000
_NPAD = 20480
_ROWS = 160
_COLS = 128
_NCLS = 80
_NIMG = 4
_S = 512
_TRASH = _NCLS * _S
_IMGSPAN = _TRASH + _NPAD
_ROWW = 16
_NC = 2
_NS = 16
_NW = _NC * _NS
_BOXW = _NIMG * _NPAD // _NW
_ZROWW = _TRASH // _NW

_NEG_INF = float("-inf")


def _shift_lanes(x, k):
    return jnp.concatenate(
        [jnp.zeros((x.shape[0], k), x.dtype), x[:, :-k]], axis=1)


def _shift_rows(x, k):
    return jnp.concatenate(
        [jnp.zeros((k, x.shape[1]), x.dtype), x[:-k, :]], axis=0)


def _score_body(p_ref, rows_ref, dst_ref):
    li = (lax.broadcasted_iota(jnp.int32, (_ROWS, _COLS), 0) * _COLS
          + lax.broadcasted_iota(jnp.int32, (_ROWS, _COLS), 1))
    for b in range(_NIMG):
        cx = p_ref[b, 0]
        cy = p_ref[b, 1]
        w = p_ref[b, 2]
        h = p_ref[b, 3]
        obj = p_ref[b, 4]
        x1 = cx - w / 2
        y1 = cy - h / 2
        x2 = cx + w / 2
        y2 = cy + h / 2
        best = p_ref[b, 5] * obj
        jbest = jnp.zeros((_ROWS, _COLS), jnp.int32)
        for c in range(1, _NCLS):
            v = p_ref[b, 5 + c] * obj
            take = v > best
            jbest = jnp.where(take, c, jbest)
            best = jnp.maximum(best, v)
        conf = best
        valid = (obj > _CONF_THRES) & (conf > _CONF_THRES)

        ustrict = (lax.broadcasted_iota(jnp.int32, (_COLS, _COLS), 0)
                   < lax.broadcasted_iota(jnp.int32, (_COLS, _COLS), 1)
                   ).astype(jnp.float32)
        tstrict = (lax.broadcasted_iota(jnp.int32, (_ROWS, _ROWS), 1)
                   < lax.broadcasted_iota(jnp.int32, (_ROWS, _ROWS), 0)
                   ).astype(jnp.float32)
        slot_lane = jnp.zeros((_ROWS, _COLS), jnp.float32)
        rowtots = []
        for c in range(_NCLS):
            m = jnp.where(valid & (jbest == c), 1.0, 0.0)
            excl_lane = jnp.dot(m, ustrict)
            slot_lane = slot_lane + jnp.where(jbest == c, excl_lane, 0.0)
            rowtots.append(jnp.sum(m, axis=1, keepdims=True))
        rmat = jnp.concatenate(rowtots, axis=1)
        excl_row_all = jnp.dot(tstrict, rmat)
        slot_row = jnp.zeros((_ROWS, _COLS), jnp.float32)
        for c in range(_NCLS):
            slot_row = slot_row + jnp.where(
                jbest == c, excl_row_all[:, c:c + 1], 0.0)
        slot = (slot_lane + slot_row).astype(jnp.int32)

        dst = jnp.where(valid & (slot < _S), jbest * _S + slot, _TRASH + li)
        rows_ref[b, 0] = conf
        rows_ref[b, 1] = x1
        rows_ref[b, 2] = y1
        rows_ref[b, 3] = x2
        rows_ref[b, 4] = y2
        rows_ref[b, 5] = li.astype(jnp.float32)
        for f in range(6, _ROWW):
            rows_ref[b, f] = jnp.zeros((_ROWS, _COLS), jnp.float32)
        dst_ref[b] = b * _IMGSPAN + dst


def _sc_scatter(rows_hbm, dst_hbm, zin_hbm, out_hbm, idx_v, rows_v, zbuf, sem):
    wid = lax.axis_index("s") * _NC + lax.axis_index("c")
    pltpu.sync_copy(zin_hbm, zbuf)
    for b in range(_NIMG):
        pltpu.sync_copy(
            zbuf, out_hbm.at[pl.ds(b * _IMGSPAN + wid * _ZROWW, _ZROWW)])
    plsc.subcore_barrier()
    base = wid * _BOXW
    pltpu.sync_copy(dst_hbm.at[pl.ds(base, _BOXW)], idx_v)
    pltpu.sync_copy(rows_hbm.at[pl.ds(base, _BOXW)], rows_v)
    pltpu.async_copy(rows_v, out_hbm.at[idx_v], sem).wait()


def _merge_body(colsc_ref, colp_ref, out_ref, sg_ref):
    lane_s = lax.broadcasted_iota(jnp.int32, (1, _S), 1)
    lane_cb = lax.broadcasted_iota(jnp.int32, (_NIMG, _COLS), 1)
    lane_sb = lax.broadcasted_iota(jnp.int32, (_NIMG, _S), 1)
    lane6 = lax.broadcasted_iota(jnp.int32, (1, 6), 1)
    big = jnp.float32(3.0e38)
    bigi = jnp.int32(2 ** 30)

    for b in range(_NIMG):
        sg_ref[b] = colsc_ref[b]

    hs0 = jnp.full((_NIMG, _COLS), _NEG_INF, jnp.float32)
    hslot0 = jnp.zeros((_NIMG, _COLS), jnp.float32)
    hidx0 = jnp.zeros((_NIMG, _COLS), jnp.float32)
    for c in range(_NCLS):
        srows = colsc_ref[:, c, :]
        m = jnp.max(srows, axis=1, keepdims=True)
        sl = jnp.min(jnp.where(srows == m, lane_sb, bigi),
                     axis=1, keepdims=True)
        idxr = colp_ref[:, 8 * c + 4, :]
        ii = jnp.sum(jnp.where(lane_sb == sl, idxr, 0.0),
                     axis=1, keepdims=True)
        upd = lane_cb == c
        hs0 = jnp.where(upd, m, hs0)
        hslot0 = jnp.where(upd, sl.astype(jnp.float32), hslot0)
        hidx0 = jnp.where(upd, ii, hidx0)

    def step(t, carry):
        hs, hslot, hidx = carry
        m4 = jnp.max(hs, axis=1, keepdims=True)
        cand = hs == m4
        mi4 = jnp.min(jnp.where(cand, hidx, big), axis=1, keepdims=True)
        csel = cand & (hidx == mi4)
        cstar4 = jnp.min(jnp.where(csel, lane_cb, bigi),
                         axis=1, keepdims=True)
        slot4 = jnp.sum(jnp.where(csel, hslot, 0.0),
                        axis=1, keepdims=True)
        ok4 = m4 > 0.0

        hs_rows = []
        hsl_rows = []
        hix_rows = []
        for b in range(_NIMG):
            cstar_v = cstar4[b:b + 1]
            cstar_s = jnp.min(cstar_v)
            okb = ok4[b:b + 1]
            mb = m4[b:b + 1]
            cstar_f = cstar_v.astype(jnp.float32)
            slot_i = slot4[b:b + 1].astype(jnp.int32)

            rows5 = colp_ref[b, pl.ds(cstar_s * 8, 8), :]
            selv = lane_s == slot_i
            w5 = jnp.sum(jnp.where(selv, rows5, 0.0),
                         axis=1, keepdims=True)
            wx1 = w5[0:1]
            wy1 = w5[1:2]
            wx2 = w5[2:3]
            wy2 = w5[3:4]

            row = jnp.where(lane6 == 0, wx1,
                  jnp.where(lane6 == 1, wy1,
                  jnp.where(lane6 == 2, wx2,
                  jnp.where(lane6 == 3, wy2,
                  jnp.where(lane6 == 4, mb, cstar_f)))))
            row = jnp.where(okb, row, 0.0)
            out_ref[b, pl.ds(t, 1), :] = row

            offs = cstar_f * _MAX_WH
            x1r = rows5[0:1]
            y1r = rows5[1:2]
            x2r = rows5[2:3]
            y2r = rows5[3:4]
            idxr = rows5[4:5]
            wbx1 = wx1 + offs
            wby1 = wy1 + offs
            wbx2 = wx2 + offs
            wby2 = wy2 + offs
            bx1 = x1r + offs
            by1 = y1r + offs
            bx2 = x2r + offs
            by2 = y2r + offs
            xx1 = jnp.maximum(wbx1, bx1)
            yy1 = jnp.maximum(wby1, by1)
            xx2 = jnp.minimum(wbx2, bx2)
            yy2 = jnp.minimum(wby2, by2)
            inter = (jnp.maximum(xx2 - xx1, 0.0)
                     * jnp.maximum(yy2 - yy1, 0.0))
            a1 = (wbx2 - wbx1) * (wby2 - wby1)
            a2 = (bx2 - bx1) * (by2 - by1)
            iou = inter / (a1 + a2 - inter + 1e-7)
            srow = sg_ref[b, pl.ds(cstar_s, 1), :]
            srow2 = jnp.where(iou > _IOU_THRES, _NEG_INF, srow)
            srow2 = jnp.where(selv, _NEG_INF, srow2)
            srow_new = jnp.where(okb, srow2, srow)
            sg_ref[b, pl.ds(cstar_s, 1), :] = srow_new

            m2 = jnp.max(srow_new, axis=1, keepdims=True)
            sl2 = jnp.min(jnp.where(srow_new == m2, lane_s, bigi),
                          axis=1, keepdims=True)
            ii2 = jnp.sum(jnp.where(lane_s == sl2, idxr, 0.0),
                          axis=1, keepdims=True)
            upd = (lane_cb[b:b + 1] == cstar_v) & okb
            hs_rows.append(jnp.where(upd, m2, hs[b:b + 1]))
            hsl_rows.append(
                jnp.where(upd, sl2.astype(jnp.float32), hslot[b:b + 1]))
            hix_rows.append(jnp.where(upd, ii2, hidx[b:b + 1]))
        return (jnp.concatenate(hs_rows, axis=0),
                jnp.concatenate(hsl_rows, axis=0),
                jnp.concatenate(hix_rows, axis=0))

    lax.fori_loop(0, _MAX_DET, step, (hs0, hslot0, hidx0))


def kernel(x):
    pred = x[0]
    pad = jnp.zeros((_NIMG, _NPAD - _N, pred.shape[-1]), pred.dtype)
    p = jnp.concatenate([pred, pad], axis=1)
    pt = p.reshape(_NIMG, _ROWS, _COLS, pred.shape[-1]).transpose(0, 3, 1, 2)

    rows, dst = pl.pallas_call(
        _score_body,
        out_shape=(
            jax.ShapeDtypeStruct((_NIMG, _ROWW, _ROWS, _COLS), jnp.float32),
            jax.ShapeDtypeStruct((_NIMG, _ROWS, _COLS), jnp.int32),
        ),
    )(pt)

    flat_rows = (rows.transpose(0, 2, 3, 1)
                 .reshape(_NIMG * _NPAD, _ROWW))
    dstf = dst.reshape(_NIMG * _NPAD)
    zin = jnp.zeros((_ZROWW, _ROWW), jnp.float32)
    scatter = functools.partial(
        pl.kernel,
        mesh=plsc.VectorSubcoreMesh(core_axis_name="c", subcore_axis_name="s"),
        compiler_params=pltpu.CompilerParams(use_tc_tiling_on_sc=False),
        out_type=jax.ShapeDtypeStruct((_NIMG * _IMGSPAN, _ROWW), jnp.float32),
        scratch_types=[
            pltpu.VMEM((_BOXW,), jnp.int32),
            pltpu.VMEM((_BOXW, _ROWW), jnp.float32),
            pltpu.VMEM((_ZROWW, _ROWW), jnp.float32),
            pltpu.SemaphoreType.DMA,
        ],
    )(_sc_scatter)
    col = scatter(flat_rows, dstf, zin).reshape(_NIMG, _IMGSPAN, _ROWW)
    colsc = col[:, :_TRASH, 0].reshape(_NIMG, _NCLS, _S)
    colp5 = (col[:, :_TRASH, 1:6]
             .reshape(_NIMG, _NCLS, _S, 5)
             .transpose(0, 1, 3, 2))
    colp = jnp.concatenate(
        [colp5, jnp.zeros((_NIMG, _NCLS, 3, _S), jnp.float32)],
        axis=2).reshape(_NIMG, _NCLS * 8, _S)

    out = pl.pallas_call(
        _merge_body,
        out_shape=jax.ShapeDtypeStruct((_NIMG, _MAX_DET, 6), jnp.float32),
        scratch_shapes=[pltpu.VMEM((_NIMG, _NCLS, _S), jnp.float32)],
    )(colsc, colp)
    return out
